# Initial kernel scaffold; baseline (speedup 1.0000x reference)
import jax
import jax.numpy as jnp
from jax.experimental import pallas as pl


def kernel(x, edge_index, W1, b1, gamma, beta, Wmu, bmu, Wls, bls):
    raise NotImplementedError("write your pallas kernel here")


# speedup vs baseline: 24.8890x; 24.8890x over previous
import functools

import jax
import jax.numpy as jnp
from jax import lax
from jax.experimental import pallas as pl
from jax.experimental.pallas import tpu as pltpu
from jax.experimental.pallas import tpu_sc as plsc

N = 10000
E = 320000
D = 128
EPS = 1e-5

NC = 2
NS = 16
NW = NC * NS
EPW = E // NW
CS = 100
CH = EPW // CS
RPT = 632
RPT_LAST = N - (NS - 1) * RPT

def _copy_owned_rows(src_ref, dst_ref, s):
    @pl.when(s < NS - 1)
    def _():
        base = s * RPT
        pltpu.sync_copy(src_ref.at[pl.ds(base, RPT)],
                        dst_ref.at[pl.ds(base, RPT)])

    @pl.when(s == NS - 1)
    def _():
        base = (NS - 1) * RPT
        pltpu.sync_copy(src_ref.at[pl.ds(base, RPT_LAST)],
                        dst_ref.at[pl.ds(base, RPT_LAST)])


def _mesh():
    return plsc.VectorSubcoreMesh(core_axis_name="c", subcore_axis_name="s",
                                  num_cores=NC, num_subcores=NS)
_HIGH = lax.Precision.HIGHEST


def _sc_degree(dst_w, ones_hbm, zeros_hbm):
    @functools.partial(
        pl.kernel,
        out_type=jax.ShapeDtypeStruct((NC, N, 8), jnp.float32),
        mesh=_mesh(),
        scratch_types=[
            pltpu.VMEM((CH, CS), jnp.int32),
            pltpu.VMEM((CS, 8), jnp.float32),
            pltpu.VMEM_SHARED((N, 8), jnp.float32),
        ],
    )
    def deg_kernel(dst_hbm, ones_h, zeros_h, out_hbm, dst_v, ones_v, acc):
        c = lax.axis_index("c")
        s = lax.axis_index("s")
        wid = s * NC + c
        _copy_owned_rows(zeros_h, acc, s)
        pltpu.sync_copy(ones_h, ones_v)
        pltpu.sync_copy(dst_hbm.at[wid], dst_v)
        plsc.subcore_barrier()

        def body(j, carry):
            pltpu.sync_copy(ones_v, acc.at[dst_v.at[j]], add=True)
            return carry

        lax.fori_loop(0, CH, body, 0)
        plsc.subcore_barrier()
        _copy_owned_rows(acc, out_hbm.at[c], s)

    return deg_kernel(dst_w, ones_hbm, zeros_hbm)


def _sc_edge_pass(table, src_w, dst_w, zeros_hbm):
    @functools.partial(
        pl.kernel,
        out_type=jax.ShapeDtypeStruct((NC, N, D), jnp.float32),
        mesh=_mesh(),
        scratch_types=[
            pltpu.VMEM((CH, CS), jnp.int32),
            pltpu.VMEM((CH, CS), jnp.int32),
            pltpu.VMEM((CS, D), jnp.float32),
            pltpu.VMEM_SHARED((N, D), jnp.float32),
            pltpu.SemaphoreType.DMA,
        ],
    )
    def pass_kernel(table_hbm, src_hbm, dst_hbm, zeros_h, out_hbm,
                    src_v, dst_v, rows_v, acc, sem):
        c = lax.axis_index("c")
        s = lax.axis_index("s")
        wid = s * NC + c
        _copy_owned_rows(zeros_h, acc, s)
        pltpu.sync_copy(src_hbm.at[wid], src_v)
        pltpu.sync_copy(dst_hbm.at[wid], dst_v)
        plsc.subcore_barrier()

        def body(j, carry):
            pltpu.async_copy(table_hbm.at[src_v.at[j]], rows_v, sem).wait()
            pltpu.sync_copy(rows_v, acc.at[dst_v.at[j]], add=True)
            return carry

        lax.fori_loop(0, CH, body, 0)
        plsc.subcore_barrier()
        _copy_owned_rows(acc, out_hbm.at[c], s)

    return pass_kernel(table, src_w, dst_w, zeros_hbm)


_BLK = 2000
_GRID = N // _BLK


def _tc_scale_matmul(x, degp, W1):
    def body(x_ref, dp_ref, w_ref, xs_ref, dv_ref):
        deg = dp_ref[0] + dp_ref[1] + 1.0
        dinv = lax.rsqrt(deg)
        xw = jnp.dot(x_ref[...], w_ref[...], precision=_HIGH,
                     preferred_element_type=jnp.float32)
        xs_ref[...] = xw * dinv[:, 0:1]
        dv_ref[...] = dinv

    return pl.pallas_call(
        body,
        grid=(_GRID,),
        in_specs=[
            pl.BlockSpec((_BLK, D), lambda i: (i, 0)),
            pl.BlockSpec((NC, _BLK, 8), lambda i: (0, i, 0)),
            pl.BlockSpec((D, D), lambda i: (0, 0)),
        ],
        out_specs=[
            pl.BlockSpec((_BLK, D), lambda i: (i, 0)),
            pl.BlockSpec((_BLK, 8), lambda i: (i, 0)),
        ],
        out_shape=[
            jax.ShapeDtypeStruct((N, D), jnp.float32),
            jax.ShapeDtypeStruct((N, 8), jnp.float32),
        ],
    )(x, degp, W1)


def _tc_combine_stats(parts, xs, dinv8, b1row):
    def body(p_ref, xs_ref, dv_ref, b_ref, hp_ref, st_ref):
        i = pl.program_id(0)
        tot = p_ref[0] + p_ref[1] + xs_ref[...]
        hp = dv_ref[:, 0:1] * tot + b_ref[...]
        hp_ref[...] = hp
        blk = jnp.concatenate(
            [jnp.sum(hp, axis=0, keepdims=True),
             jnp.sum(hp * hp, axis=0, keepdims=True),
             jnp.zeros((6, D), jnp.float32)], axis=0)

        @pl.when(i == 0)
        def _():
            st_ref[...] = blk

        @pl.when(i > 0)
        def _():
            st_ref[...] += blk

    return pl.pallas_call(
        body,
        grid=(_GRID,),
        in_specs=[
            pl.BlockSpec((NC, _BLK, D), lambda i: (0, i, 0)),
            pl.BlockSpec((_BLK, D), lambda i: (i, 0)),
            pl.BlockSpec((_BLK, 8), lambda i: (i, 0)),
            pl.BlockSpec((1, D), lambda i: (0, 0)),
        ],
        out_specs=[
            pl.BlockSpec((_BLK, D), lambda i: (i, 0)),
            pl.BlockSpec((8, D), lambda i: (0, 0)),
        ],
        out_shape=[
            jax.ShapeDtypeStruct((N, D), jnp.float32),
            jax.ShapeDtypeStruct((8, D), jnp.float32),
        ],
    )(parts, xs, dinv8, b1row)


def _tc_bn_relu_scale(hpre, stats, gammarow, betarow, dinv8):
    def body(hp_ref, st_ref, g_ref, b_ref, dv_ref, hs_ref):
        mean = st_ref[0:1] * (1.0 / N)
        var = st_ref[1:2] * (1.0 / N) - mean * mean
        rstd = lax.rsqrt(var + EPS)
        h = (hp_ref[...] - mean) * (rstd * g_ref[...]) + b_ref[...]
        h = jnp.maximum(h, 0.0)
        hs_ref[...] = h * dv_ref[:, 0:1]

    return pl.pallas_call(
        body,
        grid=(_GRID,),
        in_specs=[
            pl.BlockSpec((_BLK, D), lambda i: (i, 0)),
            pl.BlockSpec((8, D), lambda i: (0, 0)),
            pl.BlockSpec((1, D), lambda i: (0, 0)),
            pl.BlockSpec((1, D), lambda i: (0, 0)),
            pl.BlockSpec((_BLK, 8), lambda i: (i, 0)),
        ],
        out_specs=pl.BlockSpec((_BLK, D), lambda i: (i, 0)),
        out_shape=jax.ShapeDtypeStruct((N, D), jnp.float32),
    )(hpre, stats, gammarow, betarow, dinv8)


def _tc_heads(parts, hs, dinv8, Wmu, bmurow, Wls, blsrow):
    def body(p_ref, hs_ref, dv_ref, wm_ref, bm_ref, wl_ref, bl_ref,
             mu_ref, ls_ref):
        ah = dv_ref[:, 0:1] * (p_ref[0] + p_ref[1] + hs_ref[...])
        mu_ref[...] = jnp.dot(ah, wm_ref[...], precision=_HIGH,
                              preferred_element_type=jnp.float32) + bm_ref[...]
        ls_ref[...] = jnp.dot(ah, wl_ref[...], precision=_HIGH,
                              preferred_element_type=jnp.float32) + bl_ref[...]

    return pl.pallas_call(
        body,
        grid=(_GRID,),
        in_specs=[
            pl.BlockSpec((NC, _BLK, D), lambda i: (0, i, 0)),
            pl.BlockSpec((_BLK, D), lambda i: (i, 0)),
            pl.BlockSpec((_BLK, 8), lambda i: (i, 0)),
            pl.BlockSpec((D, D), lambda i: (0, 0)),
            pl.BlockSpec((1, D), lambda i: (0, 0)),
            pl.BlockSpec((D, D), lambda i: (0, 0)),
            pl.BlockSpec((1, D), lambda i: (0, 0)),
        ],
        out_specs=[
            pl.BlockSpec((_BLK, D), lambda i: (i, 0)),
            pl.BlockSpec((_BLK, D), lambda i: (i, 0)),
        ],
        out_shape=[
            jax.ShapeDtypeStruct((N, D), jnp.float32),
            jax.ShapeDtypeStruct((N, D), jnp.float32),
        ],
    )(parts, hs, dinv8, Wmu, bmurow, Wls, blsrow)


def kernel(x, edge_index, W1, b1, gamma, beta, Wmu, bmu, Wls, bls):
    src_w = edge_index[0].astype(jnp.int32).reshape(NW, CH, CS)
    dst_w = edge_index[1].astype(jnp.int32).reshape(NW, CH, CS)
    zeros_d = jnp.zeros((N, D), jnp.float32)
    zeros_8 = jnp.zeros((N, 8), jnp.float32)
    ones_8 = jnp.ones((CS, 8), jnp.float32)
    b1row = b1.reshape(1, D)

    degp = _sc_degree(dst_w, ones_8, zeros_8)
    xs, dinv8 = _tc_scale_matmul(x, degp, W1)
    parts1 = _sc_edge_pass(xs, src_w, dst_w, zeros_d)
    hpre, stats = _tc_combine_stats(parts1, xs, dinv8, b1row)
    hs = _tc_bn_relu_scale(hpre, stats, gamma.reshape(1, D),
                           beta.reshape(1, D), dinv8)
    parts2 = _sc_edge_pass(hs, src_w, dst_w, zeros_d)
    mu, ls = _tc_heads(parts2, hs, dinv8, Wmu, bmu.reshape(1, D),
                       Wls, bls.reshape(1, D))
    return (mu, ls)


# speedup vs baseline: 33.0482x; 1.3278x over previous
import functools

import jax
import jax.numpy as jnp
from jax import lax
from jax.experimental import pallas as pl
from jax.experimental.pallas import tpu as pltpu
from jax.experimental.pallas import tpu_sc as plsc

N = 10000
E = 320000
D = 128
EPS = 1e-5

NC = 2
NS = 16
NW = NC * NS
EPW = E // NW
CS = 125
CH = EPW // CS
GB = 8
NG = CH // GB
RPT = 632
RPT_LAST = N - (NS - 1) * RPT

def _copy_owned_rows(src_ref, dst_ref, s):
    @pl.when(s < NS - 1)
    def _():
        base = s * RPT
        pltpu.sync_copy(src_ref.at[pl.ds(base, RPT)],
                        dst_ref.at[pl.ds(base, RPT)])

    @pl.when(s == NS - 1)
    def _():
        base = (NS - 1) * RPT
        pltpu.sync_copy(src_ref.at[pl.ds(base, RPT_LAST)],
                        dst_ref.at[pl.ds(base, RPT_LAST)])


def _mesh():
    return plsc.VectorSubcoreMesh(core_axis_name="c", subcore_axis_name="s",
                                  num_cores=NC, num_subcores=NS)
_HIGH = lax.Precision.HIGHEST


def _sc_degree(dst_w, ones_hbm, zeros_hbm):
    @functools.partial(
        pl.kernel,
        out_type=jax.ShapeDtypeStruct((NC, N, 8), jnp.float32),
        mesh=_mesh(),
        scratch_types=[
            pltpu.VMEM((CH, CS), jnp.int32),
            pltpu.VMEM((CS, 8), jnp.float32),
            pltpu.VMEM_SHARED((N, 8), jnp.float32),
        ],
    )
    def deg_kernel(dst_hbm, ones_h, zeros_h, out_hbm, dst_v, ones_v, acc):
        c = lax.axis_index("c")
        s = lax.axis_index("s")
        wid = s * NC + c
        _copy_owned_rows(zeros_h, acc, s)
        pltpu.sync_copy(ones_h, ones_v)
        pltpu.sync_copy(dst_hbm.at[wid], dst_v)
        plsc.subcore_barrier()

        def body(j, carry):
            pltpu.sync_copy(ones_v, acc.at[dst_v.at[j]], add=True)
            return carry

        lax.fori_loop(0, CH, body, 0)
        plsc.subcore_barrier()
        _copy_owned_rows(acc, out_hbm.at[c], s)

    return deg_kernel(dst_w, ones_hbm, zeros_hbm)


def _sc_edge_pass(table, src_w, dst_w, zeros_hbm):
    @functools.partial(
        pl.kernel,
        out_type=jax.ShapeDtypeStruct((NC, N, D), jnp.float32),
        mesh=_mesh(),
        scratch_types=[
            pltpu.VMEM((GB, CS), jnp.int32),
            pltpu.VMEM((GB, CS), jnp.int32),
            pltpu.VMEM((CS, D), jnp.float32),
            pltpu.VMEM((CS, D), jnp.float32),
            pltpu.VMEM_SHARED((N, D), jnp.float32),
            pltpu.SemaphoreType.DMA,
            pltpu.SemaphoreType.DMA,
        ],
    )
    def pass_kernel(table_hbm, src_hbm, dst_hbm, zeros_h, out_hbm,
                    src_v, dst_v, rows0, rows1, acc, sem0, sem1):
        c = lax.axis_index("c")
        s = lax.axis_index("s")
        wid = s * NC + c
        _copy_owned_rows(zeros_h, acc, s)
        plsc.subcore_barrier()

        def group(g, carry):
            pltpu.sync_copy(src_hbm.at[wid].at[pl.ds(g * GB, GB)], src_v)
            pltpu.sync_copy(dst_hbm.at[wid].at[pl.ds(g * GB, GB)], dst_v)
            pltpu.async_copy(table_hbm.at[src_v.at[0]], rows0, sem0)
            for k in range(0, GB, 2):
                pltpu.async_copy(table_hbm.at[src_v.at[k + 1]], rows1, sem1)
                pltpu.make_async_copy(table_hbm.at[src_v.at[k]], rows0,
                                      sem0).wait()
                pltpu.sync_copy(rows0, acc.at[dst_v.at[k]], add=True)
                if k + 2 < GB:
                    pltpu.async_copy(table_hbm.at[src_v.at[k + 2]], rows0,
                                     sem0)
                pltpu.make_async_copy(table_hbm.at[src_v.at[k + 1]], rows1,
                                      sem1).wait()
                pltpu.sync_copy(rows1, acc.at[dst_v.at[k + 1]], add=True)
            return carry

        lax.fori_loop(0, NG, group, 0)
        plsc.subcore_barrier()
        _copy_owned_rows(acc, out_hbm.at[c], s)

    return pass_kernel(table, src_w, dst_w, zeros_hbm)


_BLK = 2000
_GRID = N // _BLK


def _tc_scale_matmul(x, degp, W1):
    def body(x_ref, dp_ref, w_ref, xs_ref, dv_ref):
        deg = dp_ref[0] + dp_ref[1] + 1.0
        dinv = lax.rsqrt(deg)
        xw = jnp.dot(x_ref[...], w_ref[...], precision=_HIGH,
                     preferred_element_type=jnp.float32)
        xs_ref[...] = xw * dinv[:, 0:1]
        dv_ref[...] = dinv

    return pl.pallas_call(
        body,
        grid=(_GRID,),
        in_specs=[
            pl.BlockSpec((_BLK, D), lambda i: (i, 0)),
            pl.BlockSpec((NC, _BLK, 8), lambda i: (0, i, 0)),
            pl.BlockSpec((D, D), lambda i: (0, 0)),
        ],
        out_specs=[
            pl.BlockSpec((_BLK, D), lambda i: (i, 0)),
            pl.BlockSpec((_BLK, 8), lambda i: (i, 0)),
        ],
        out_shape=[
            jax.ShapeDtypeStruct((N, D), jnp.float32),
            jax.ShapeDtypeStruct((N, 8), jnp.float32),
        ],
    )(x, degp, W1)


def _tc_combine_stats(parts, xs, dinv8, b1row):
    def body(p_ref, xs_ref, dv_ref, b_ref, hp_ref, st_ref):
        i = pl.program_id(0)
        tot = p_ref[0] + p_ref[1] + xs_ref[...]
        hp = dv_ref[:, 0:1] * tot + b_ref[...]
        hp_ref[...] = hp
        blk = jnp.concatenate(
            [jnp.sum(hp, axis=0, keepdims=True),
             jnp.sum(hp * hp, axis=0, keepdims=True),
             jnp.zeros((6, D), jnp.float32)], axis=0)

        @pl.when(i == 0)
        def _():
            st_ref[...] = blk

        @pl.when(i > 0)
        def _():
            st_ref[...] += blk

    return pl.pallas_call(
        body,
        grid=(_GRID,),
        in_specs=[
            pl.BlockSpec((NC, _BLK, D), lambda i: (0, i, 0)),
            pl.BlockSpec((_BLK, D), lambda i: (i, 0)),
            pl.BlockSpec((_BLK, 8), lambda i: (i, 0)),
            pl.BlockSpec((1, D), lambda i: (0, 0)),
        ],
        out_specs=[
            pl.BlockSpec((_BLK, D), lambda i: (i, 0)),
            pl.BlockSpec((8, D), lambda i: (0, 0)),
        ],
        out_shape=[
            jax.ShapeDtypeStruct((N, D), jnp.float32),
            jax.ShapeDtypeStruct((8, D), jnp.float32),
        ],
    )(parts, xs, dinv8, b1row)


def _tc_bn_relu_scale(hpre, stats, gammarow, betarow, dinv8):
    def body(hp_ref, st_ref, g_ref, b_ref, dv_ref, hs_ref):
        mean = st_ref[0:1] * (1.0 / N)
        var = st_ref[1:2] * (1.0 / N) - mean * mean
        rstd = lax.rsqrt(var + EPS)
        h = (hp_ref[...] - mean) * (rstd * g_ref[...]) + b_ref[...]
        h = jnp.maximum(h, 0.0)
        hs_ref[...] = h * dv_ref[:, 0:1]

    return pl.pallas_call(
        body,
        grid=(_GRID,),
        in_specs=[
            pl.BlockSpec((_BLK, D), lambda i: (i, 0)),
            pl.BlockSpec((8, D), lambda i: (0, 0)),
            pl.BlockSpec((1, D), lambda i: (0, 0)),
            pl.BlockSpec((1, D), lambda i: (0, 0)),
            pl.BlockSpec((_BLK, 8), lambda i: (i, 0)),
        ],
        out_specs=pl.BlockSpec((_BLK, D), lambda i: (i, 0)),
        out_shape=jax.ShapeDtypeStruct((N, D), jnp.float32),
    )(hpre, stats, gammarow, betarow, dinv8)


def _tc_heads(parts, hs, dinv8, Wmu, bmurow, Wls, blsrow):
    def body(p_ref, hs_ref, dv_ref, wm_ref, bm_ref, wl_ref, bl_ref,
             mu_ref, ls_ref):
        ah = dv_ref[:, 0:1] * (p_ref[0] + p_ref[1] + hs_ref[...])
        mu_ref[...] = jnp.dot(ah, wm_ref[...], precision=_HIGH,
                              preferred_element_type=jnp.float32) + bm_ref[...]
        ls_ref[...] = jnp.dot(ah, wl_ref[...], precision=_HIGH,
                              preferred_element_type=jnp.float32) + bl_ref[...]

    return pl.pallas_call(
        body,
        grid=(_GRID,),
        in_specs=[
            pl.BlockSpec((NC, _BLK, D), lambda i: (0, i, 0)),
            pl.BlockSpec((_BLK, D), lambda i: (i, 0)),
            pl.BlockSpec((_BLK, 8), lambda i: (i, 0)),
            pl.BlockSpec((D, D), lambda i: (0, 0)),
            pl.BlockSpec((1, D), lambda i: (0, 0)),
            pl.BlockSpec((D, D), lambda i: (0, 0)),
            pl.BlockSpec((1, D), lambda i: (0, 0)),
        ],
        out_specs=[
            pl.BlockSpec((_BLK, D), lambda i: (i, 0)),
            pl.BlockSpec((_BLK, D), lambda i: (i, 0)),
        ],
        out_shape=[
            jax.ShapeDtypeStruct((N, D), jnp.float32),
            jax.ShapeDtypeStruct((N, D), jnp.float32),
        ],
    )(parts, hs, dinv8, Wmu, bmurow, Wls, blsrow)


def kernel(x, edge_index, W1, b1, gamma, beta, Wmu, bmu, Wls, bls):
    src_w = edge_index[0].astype(jnp.int32).reshape(NW, CH, CS)
    dst_w = edge_index[1].astype(jnp.int32).reshape(NW, CH, CS)
    zeros_d = jnp.zeros((N, D), jnp.float32)
    zeros_8 = jnp.zeros((N, 8), jnp.float32)
    ones_8 = jnp.ones((CS, 8), jnp.float32)
    b1row = b1.reshape(1, D)

    degp = _sc_degree(dst_w, ones_8, zeros_8)
    xs, dinv8 = _tc_scale_matmul(x, degp, W1)
    parts1 = _sc_edge_pass(xs, src_w, dst_w, zeros_d)
    hpre, stats = _tc_combine_stats(parts1, xs, dinv8, b1row)
    hs = _tc_bn_relu_scale(hpre, stats, gamma.reshape(1, D),
                           beta.reshape(1, D), dinv8)
    parts2 = _sc_edge_pass(hs, src_w, dst_w, zeros_d)
    mu, ls = _tc_heads(parts2, hs, dinv8, Wmu, bmu.reshape(1, D),
                       Wls, bls.reshape(1, D))
    return (mu, ls)


# speedup vs baseline: 35.2817x; 1.0676x over previous
import functools

import jax
import jax.numpy as jnp
from jax import lax
from jax.experimental import pallas as pl
from jax.experimental.pallas import tpu as pltpu
from jax.experimental.pallas import tpu_sc as plsc

N = 10000
E = 320000
D = 128
EPS = 1e-5

NC = 2
NS = 16
NW = NC * NS
EPW = E // NW
CS = 125
CH = EPW // CS
GB = 8
NG = CH // GB
RPT = 632
RPT_LAST = N - (NS - 1) * RPT

def _copy_owned_rows(src_ref, dst_ref, s):
    @pl.when(s < NS - 1)
    def _():
        base = s * RPT
        pltpu.sync_copy(src_ref.at[pl.ds(base, RPT)],
                        dst_ref.at[pl.ds(base, RPT)])

    @pl.when(s == NS - 1)
    def _():
        base = (NS - 1) * RPT
        pltpu.sync_copy(src_ref.at[pl.ds(base, RPT_LAST)],
                        dst_ref.at[pl.ds(base, RPT_LAST)])


def _mesh():
    return plsc.VectorSubcoreMesh(core_axis_name="c", subcore_axis_name="s",
                                  num_cores=NC, num_subcores=NS)
_HIGH = lax.Precision.HIGHEST


def _sc_degree(dst_w, ones_hbm, zeros_hbm):
    @functools.partial(
        pl.kernel,
        out_type=jax.ShapeDtypeStruct((NC, N, 8), jnp.float32),
        mesh=_mesh(),
        scratch_types=[
            pltpu.VMEM((CH, CS), jnp.int32),
            pltpu.VMEM((CS, 8), jnp.float32),
            pltpu.VMEM_SHARED((N, 8), jnp.float32),
        ],
    )
    def deg_kernel(dst_hbm, ones_h, zeros_h, out_hbm, dst_v, ones_v, acc):
        c = lax.axis_index("c")
        s = lax.axis_index("s")
        wid = s * NC + c
        _copy_owned_rows(zeros_h, acc, s)
        pltpu.sync_copy(ones_h, ones_v)
        pltpu.sync_copy(dst_hbm.at[wid], dst_v)
        plsc.subcore_barrier()

        def body(j, carry):
            pltpu.sync_copy(ones_v, acc.at[dst_v.at[j]], add=True)
            return carry

        lax.fori_loop(0, CH, body, 0)
        plsc.subcore_barrier()
        _copy_owned_rows(acc, out_hbm.at[c], s)

    return deg_kernel(dst_w, ones_hbm, zeros_hbm)


def _sc_edge_pass(table, src_w, dst_w, zeros_hbm):
    @functools.partial(
        pl.kernel,
        out_type=jax.ShapeDtypeStruct((NC, N, D), jnp.float32),
        mesh=_mesh(),
        scratch_types=[
            pltpu.VMEM((GB, CS), jnp.int32),
            pltpu.VMEM((GB, CS), jnp.int32),
            pltpu.VMEM((GB, CS), jnp.int32),
            pltpu.VMEM((GB, CS), jnp.int32),
            pltpu.VMEM((CS, D), jnp.float32),
            pltpu.VMEM((CS, D), jnp.float32),
            pltpu.VMEM_SHARED((N, D), jnp.float32),
            pltpu.SemaphoreType.DMA,
            pltpu.SemaphoreType.DMA,
            pltpu.SemaphoreType.DMA,
        ],
    )
    def pass_kernel(table_hbm, src_hbm, dst_hbm, zeros_h, out_hbm,
                    src_v0, dst_v0, src_v1, dst_v1, rows0, rows1, acc,
                    sem0, sem1, semr):
        c = lax.axis_index("c")
        s = lax.axis_index("s")
        wid = s * NC + c

        def refill(g, sv, dv):
            pltpu.async_copy(src_hbm.at[wid].at[pl.ds(g * GB, GB)], sv, semr)
            pltpu.async_copy(dst_hbm.at[wid].at[pl.ds(g * GB, GB)], dv, semr)

        def wait_refill(g, sv, dv):
            pltpu.make_async_copy(src_hbm.at[wid].at[pl.ds(g * GB, GB)],
                                  sv, semr).wait()
            pltpu.make_async_copy(dst_hbm.at[wid].at[pl.ds(g * GB, GB)],
                                  dv, semr).wait()

        def chunks(sv, dv):
            pltpu.async_copy(table_hbm.at[sv.at[0]], rows0, sem0)
            for k in range(0, GB, 2):
                pltpu.async_copy(table_hbm.at[sv.at[k + 1]], rows1, sem1)
                pltpu.make_async_copy(table_hbm.at[sv.at[k]], rows0,
                                      sem0).wait()
                pltpu.sync_copy(rows0, acc.at[dv.at[k]], add=True)
                if k + 2 < GB:
                    pltpu.async_copy(table_hbm.at[sv.at[k + 2]], rows0,
                                     sem0)
                pltpu.make_async_copy(table_hbm.at[sv.at[k + 1]], rows1,
                                      sem1).wait()
                pltpu.sync_copy(rows1, acc.at[dv.at[k + 1]], add=True)

        refill(0, src_v0, dst_v0)
        _copy_owned_rows(zeros_h, acc, s)
        plsc.subcore_barrier()

        def group_pair(i, carry):
            g = i * 2
            refill(g + 1, src_v1, dst_v1)
            wait_refill(g, src_v0, dst_v0)
            chunks(src_v0, dst_v0)
            refill(jnp.minimum(g + 2, NG - 1), src_v0, dst_v0)
            wait_refill(g + 1, src_v1, dst_v1)
            chunks(src_v1, dst_v1)
            return carry

        lax.fori_loop(0, NG // 2, group_pair, 0)
        wait_refill(NG - 1, src_v0, dst_v0)
        plsc.subcore_barrier()
        _copy_owned_rows(acc, out_hbm.at[c], s)

    return pass_kernel(table, src_w, dst_w, zeros_hbm)


_BLK = 2000
_GRID = N // _BLK


def _tc_scale_matmul(x, degp, W1):
    def body(x_ref, dp_ref, w_ref, xs_ref, dv_ref):
        deg = dp_ref[0] + dp_ref[1] + 1.0
        dinv = lax.rsqrt(deg)
        xw = jnp.dot(x_ref[...], w_ref[...], precision=_HIGH,
                     preferred_element_type=jnp.float32)
        xs_ref[...] = xw * dinv[:, 0:1]
        dv_ref[...] = dinv

    return pl.pallas_call(
        body,
        grid=(_GRID,),
        in_specs=[
            pl.BlockSpec((_BLK, D), lambda i: (i, 0)),
            pl.BlockSpec((NC, _BLK, 8), lambda i: (0, i, 0)),
            pl.BlockSpec((D, D), lambda i: (0, 0)),
        ],
        out_specs=[
            pl.BlockSpec((_BLK, D), lambda i: (i, 0)),
            pl.BlockSpec((_BLK, 8), lambda i: (i, 0)),
        ],
        out_shape=[
            jax.ShapeDtypeStruct((N, D), jnp.float32),
            jax.ShapeDtypeStruct((N, 8), jnp.float32),
        ],
    )(x, degp, W1)


def _tc_combine_stats(parts, xs, dinv8, b1row):
    def body(p_ref, xs_ref, dv_ref, b_ref, hp_ref, st_ref):
        i = pl.program_id(0)
        tot = p_ref[0] + p_ref[1] + xs_ref[...]
        hp = dv_ref[:, 0:1] * tot + b_ref[...]
        hp_ref[...] = hp
        blk = jnp.concatenate(
            [jnp.sum(hp, axis=0, keepdims=True),
             jnp.sum(hp * hp, axis=0, keepdims=True),
             jnp.zeros((6, D), jnp.float32)], axis=0)

        @pl.when(i == 0)
        def _():
            st_ref[...] = blk

        @pl.when(i > 0)
        def _():
            st_ref[...] += blk

    return pl.pallas_call(
        body,
        grid=(_GRID,),
        in_specs=[
            pl.BlockSpec((NC, _BLK, D), lambda i: (0, i, 0)),
            pl.BlockSpec((_BLK, D), lambda i: (i, 0)),
            pl.BlockSpec((_BLK, 8), lambda i: (i, 0)),
            pl.BlockSpec((1, D), lambda i: (0, 0)),
        ],
        out_specs=[
            pl.BlockSpec((_BLK, D), lambda i: (i, 0)),
            pl.BlockSpec((8, D), lambda i: (0, 0)),
        ],
        out_shape=[
            jax.ShapeDtypeStruct((N, D), jnp.float32),
            jax.ShapeDtypeStruct((8, D), jnp.float32),
        ],
    )(parts, xs, dinv8, b1row)


def _tc_bn_relu_scale(hpre, stats, gammarow, betarow, dinv8):
    def body(hp_ref, st_ref, g_ref, b_ref, dv_ref, hs_ref):
        mean = st_ref[0:1] * (1.0 / N)
        var = st_ref[1:2] * (1.0 / N) - mean * mean
        rstd = lax.rsqrt(var + EPS)
        h = (hp_ref[...] - mean) * (rstd * g_ref[...]) + b_ref[...]
        h = jnp.maximum(h, 0.0)
        hs_ref[...] = h * dv_ref[:, 0:1]

    return pl.pallas_call(
        body,
        grid=(_GRID,),
        in_specs=[
            pl.BlockSpec((_BLK, D), lambda i: (i, 0)),
            pl.BlockSpec((8, D), lambda i: (0, 0)),
            pl.BlockSpec((1, D), lambda i: (0, 0)),
            pl.BlockSpec((1, D), lambda i: (0, 0)),
            pl.BlockSpec((_BLK, 8), lambda i: (i, 0)),
        ],
        out_specs=pl.BlockSpec((_BLK, D), lambda i: (i, 0)),
        out_shape=jax.ShapeDtypeStruct((N, D), jnp.float32),
    )(hpre, stats, gammarow, betarow, dinv8)


def _tc_heads(parts, hs, dinv8, Wmu, bmurow, Wls, blsrow):
    def body(p_ref, hs_ref, dv_ref, wm_ref, bm_ref, wl_ref, bl_ref,
             mu_ref, ls_ref):
        ah = dv_ref[:, 0:1] * (p_ref[0] + p_ref[1] + hs_ref[...])
        mu_ref[...] = jnp.dot(ah, wm_ref[...], precision=_HIGH,
                              preferred_element_type=jnp.float32) + bm_ref[...]
        ls_ref[...] = jnp.dot(ah, wl_ref[...], precision=_HIGH,
                              preferred_element_type=jnp.float32) + bl_ref[...]

    return pl.pallas_call(
        body,
        grid=(_GRID,),
        in_specs=[
            pl.BlockSpec((NC, _BLK, D), lambda i: (0, i, 0)),
            pl.BlockSpec((_BLK, D), lambda i: (i, 0)),
            pl.BlockSpec((_BLK, 8), lambda i: (i, 0)),
            pl.BlockSpec((D, D), lambda i: (0, 0)),
            pl.BlockSpec((1, D), lambda i: (0, 0)),
            pl.BlockSpec((D, D), lambda i: (0, 0)),
            pl.BlockSpec((1, D), lambda i: (0, 0)),
        ],
        out_specs=[
            pl.BlockSpec((_BLK, D), lambda i: (i, 0)),
            pl.BlockSpec((_BLK, D), lambda i: (i, 0)),
        ],
        out_shape=[
            jax.ShapeDtypeStruct((N, D), jnp.float32),
            jax.ShapeDtypeStruct((N, D), jnp.float32),
        ],
    )(parts, hs, dinv8, Wmu, bmurow, Wls, blsrow)


def kernel(x, edge_index, W1, b1, gamma, beta, Wmu, bmu, Wls, bls):
    src_w = edge_index[0].astype(jnp.int32).reshape(NW, CH, CS)
    dst_w = edge_index[1].astype(jnp.int32).reshape(NW, CH, CS)
    zeros_d = jnp.zeros((N, D), jnp.float32)
    zeros_8 = jnp.zeros((N, 8), jnp.float32)
    ones_8 = jnp.ones((CS, 8), jnp.float32)
    b1row = b1.reshape(1, D)

    degp = _sc_degree(dst_w, ones_8, zeros_8)
    xs, dinv8 = _tc_scale_matmul(x, degp, W1)
    parts1 = _sc_edge_pass(xs, src_w, dst_w, zeros_d)
    hpre, stats = _tc_combine_stats(parts1, xs, dinv8, b1row)
    hs = _tc_bn_relu_scale(hpre, stats, gamma.reshape(1, D),
                           beta.reshape(1, D), dinv8)
    parts2 = _sc_edge_pass(hs, src_w, dst_w, zeros_d)
    mu, ls = _tc_heads(parts2, hs, dinv8, Wmu, bmu.reshape(1, D),
                       Wls, bls.reshape(1, D))
    return (mu, ls)
